# baseline (speedup 1.0000x reference)
import functools

import jax
import jax.numpy as jnp
from jax import lax
from jax.experimental import pallas as pl
from jax.experimental.pallas import tpu as pltpu
from jax.experimental.pallas import tpu_sc as plsc

N = 10000
N_PAD = 10240
E = 320000
E2 = E + N
HALF = 32
K = 5
ALPHA = 0.1
B = 128
NBUF = 4
NB = 168
CB = 28
NCB = NB // CB
CBT = CB // NBUF
E_PAD = 16 * NB * B
RPT = N_PAD // 16

_mesh = plsc.VectorSubcoreMesh(core_axis_name="c", subcore_axis_name="s")


def _mlp_body(x_ref, w1_ref, b1_ref, w2_ref, b2_ref, o_ref):
    h = jnp.maximum(
        jnp.dot(x_ref[...], w1_ref[...], preferred_element_type=jnp.float32)
        + b1_ref[...],
        0.0,
    )
    o_ref[0] = (
        jnp.dot(h, w2_ref[0], preferred_element_type=jnp.float32) + b2_ref[0]
    )


def _mlp(x_pad, w1t, b1, w2t, b2):
    blk = 1024
    return pl.pallas_call(
        _mlp_body,
        grid=(2, N_PAD // blk),
        in_specs=[
            pl.BlockSpec((blk, 128), lambda c, i: (i, 0)),
            pl.BlockSpec((128, 128), lambda c, i: (0, 0)),
            pl.BlockSpec((1, 128), lambda c, i: (0, 0)),
            pl.BlockSpec((1, 128, HALF), lambda c, i: (c, 0, 0)),
            pl.BlockSpec((1, 1, HALF), lambda c, i: (c, 0, 0)),
        ],
        out_specs=pl.BlockSpec((1, blk, HALF), lambda c, i: (c, i, 0)),
        out_shape=jax.ShapeDtypeStruct((2, N_PAD, HALF), jnp.float32),
    )(x_pad, w1t, b1, w2t, b2)


def _rsqrt16(x):
    i = lax.bitcast_convert_type(x, jnp.int32)
    i = jnp.int32(0x5F3759DF) - (i >> 1)
    y = lax.bitcast_convert_type(i, jnp.float32)
    for _ in range(4):
        y = y * (1.5 - 0.5 * x * y * y)
    return y


def _appnp_body(h_hbm, src_hbm, dst_hbm, out_hbm,
                src_v, dst_v, b0, b1, b2, b3, ones_v, dinv_v, c2_v,
                strip_v, s_sp, u_sp, m0, m1, m2, m3):
    bufs = (b0, b1, b2, b3)
    sems = (m0, m1, m2, m3)
    c = lax.axis_index("c")
    s = lax.axis_index("s")
    r0 = s * RPT
    ubase = c * N_PAD + r0

    one16 = jnp.full((16,), 1.0, jnp.float32)
    zero16 = jnp.zeros((16,), jnp.float32)

    def fill_ones(j, carry):
        ones_v[j, pl.ds(0, 16)] = one16
        ones_v[j, pl.ds(16, 16)] = one16
        return carry

    lax.fori_loop(0, B, fill_ones, 0, unroll=False)

    def zero_strip(r, carry):
        strip_v[r, pl.ds(0, 16)] = zero16
        strip_v[r, pl.ds(16, 16)] = zero16
        return carry

    lax.fori_loop(0, RPT, zero_strip, 0, unroll=False)

    pltpu.sync_copy(strip_v, s_sp.at[pl.ds(r0, RPT)])
    plsc.subcore_barrier()

    def degchunk(q, carry):
        pltpu.sync_copy(dst_hbm.at[s, pl.ds(q * CB, CB)], dst_v)

        def degblk(j, cr):
            pltpu.sync_copy(ones_v, s_sp.at[dst_v.at[j]], add=True)
            return cr

        lax.fori_loop(0, CB, degblk, 0, unroll=False)
        return carry

    lax.fori_loop(0, NCB, degchunk, 0, unroll=False)
    plsc.subcore_barrier()

    pltpu.sync_copy(s_sp.at[pl.ds(r0, RPT)], strip_v)

    def zero_c2(r, carry):
        c2_v[r, pl.ds(0, 16)] = zero16
        c2_v[r, pl.ds(16, 16)] = zero16
        return carry

    lax.fori_loop(0, RPT, zero_c2, 0, unroll=False)
    pltpu.sync_copy(c2_v, s_sp.at[pl.ds(r0, RPT)])
    pltpu.sync_copy(h_hbm.at[pl.ds(ubase, RPT)], c2_v)

    def setup_row(r, carry):
        d = _rsqrt16(strip_v[r, pl.ds(0, 16)])
        dinv_v[r] = d
        h0 = c2_v[r, pl.ds(0, 16)]
        h1 = c2_v[r, pl.ds(16, 16)]
        c2_v[r, pl.ds(0, 16)] = ALPHA * h0
        c2_v[r, pl.ds(16, 16)] = ALPHA * h1
        strip_v[r, pl.ds(0, 16)] = d * h0
        strip_v[r, pl.ds(16, 16)] = d * h1
        return carry

    lax.fori_loop(0, RPT, setup_row, 0, unroll=False)
    pltpu.sync_copy(strip_v, u_sp.at[pl.ds(r0, RPT)])
    plsc.subcore_barrier()

    def hop(k, carry):
        def chunk(q, cq):
            pltpu.sync_copy(src_hbm.at[s, pl.ds(q * CB, CB)], src_v)
            pltpu.sync_copy(dst_hbm.at[s, pl.ds(q * CB, CB)], dst_v)
            for b in range(NBUF):
                pltpu.async_copy(u_sp.at[src_v.at[b]], bufs[b], sems[b])

            def blk(t, cr):
                base = t * NBUF
                for b in range(NBUF):
                    j = base + b
                    pltpu.make_async_copy(u_sp.at[src_v.at[j]], bufs[b],
                                          sems[b]).wait()
                    pltpu.sync_copy(bufs[b], s_sp.at[dst_v.at[j]], add=True)
                    pltpu.async_copy(u_sp.at[src_v.at[j + NBUF]], bufs[b],
                                     sems[b])
                return cr

            lax.fori_loop(0, CBT - 1, blk, 0, unroll=False)
            base = (CBT - 1) * NBUF
            for b in range(NBUF):
                j = base + b
                pltpu.make_async_copy(u_sp.at[src_v.at[j]], bufs[b],
                                      sems[b]).wait()
                pltpu.sync_copy(bufs[b], s_sp.at[dst_v.at[j]], add=True)
            return cq

        lax.fori_loop(0, NCB, chunk, 0, unroll=False)
        plsc.subcore_barrier()

        pltpu.sync_copy(s_sp.at[pl.ds(r0, RPT)], strip_v)
        last = k == K - 1

        def comb(r, cr):
            d = dinv_v[r]
            s0 = strip_v[r, pl.ds(0, 16)]
            s1 = strip_v[r, pl.ds(16, 16)]
            e0 = c2_v[r, pl.ds(0, 16)]
            e1 = c2_v[r, pl.ds(16, 16)]
            ds0 = (1.0 - ALPHA) * d * s0
            ds1 = (1.0 - ALPHA) * d * s1
            strip_v[r, pl.ds(0, 16)] = jnp.where(last, ds0 + e0,
                                                 d * (ds0 + e0))
            strip_v[r, pl.ds(16, 16)] = jnp.where(last, ds1 + e1,
                                                  d * (ds1 + e1))
            return cr

        lax.fori_loop(0, RPT, comb, 0, unroll=False)

        pltpu.sync_copy(strip_v, u_sp.at[pl.ds(r0, RPT)])

        @pl.when(last)
        def _():
            pltpu.sync_copy(strip_v, out_hbm.at[c, pl.ds(r0, RPT)])

        lax.fori_loop(0, RPT, zero_strip, 0, unroll=False)
        pltpu.sync_copy(strip_v, s_sp.at[pl.ds(r0, RPT)])
        plsc.subcore_barrier()
        return carry

    lax.fori_loop(0, K, hop, 0, unroll=False)


@functools.partial(
    pl.kernel,
    out_type=jax.ShapeDtypeStruct((2, N_PAD, HALF), jnp.float32),
    mesh=_mesh,
    compiler_params=pltpu.CompilerParams(use_tc_tiling_on_sc=False),
    scratch_types=[
        pltpu.VMEM((CB, B), jnp.int32),
        pltpu.VMEM((CB, B), jnp.int32),
        pltpu.VMEM((B, HALF), jnp.float32),
        pltpu.VMEM((B, HALF), jnp.float32),
        pltpu.VMEM((B, HALF), jnp.float32),
        pltpu.VMEM((B, HALF), jnp.float32),
        pltpu.VMEM((B, HALF), jnp.float32),
        pltpu.VMEM((RPT, 16), jnp.float32),
        pltpu.VMEM((RPT, HALF), jnp.float32),
        pltpu.VMEM((RPT, HALF), jnp.float32),
        pltpu.VMEM_SHARED((N_PAD, HALF), jnp.float32),
        pltpu.VMEM_SHARED((N_PAD, HALF), jnp.float32),
        pltpu.SemaphoreType.DMA,
        pltpu.SemaphoreType.DMA,
        pltpu.SemaphoreType.DMA,
        pltpu.SemaphoreType.DMA,
    ],
)
def _appnp_sc(h_hbm, src_hbm, dst_hbm, out_hbm,
              src_v, dst_v, b0, b1, b2, b3, ones_v, dinv_v, c2_v,
              strip_v, s_sp, u_sp, m0, m1, m2, m3):
    _appnp_body(h_hbm, src_hbm, dst_hbm, out_hbm,
                src_v, dst_v, b0, b1, b2, b3, ones_v, dinv_v, c2_v,
                strip_v, s_sp, u_sp, m0, m1, m2, m3)


def kernel(x, edge_index, W1, b1, W2, b2):
    loop = jnp.arange(N, dtype=jnp.int32)
    src = jnp.concatenate([edge_index[0], loop])
    dst = jnp.concatenate([edge_index[1], loop])
    pad = E_PAD - E2
    pad_idx = jnp.arange(pad, dtype=jnp.int32)
    src_p = jnp.concatenate([src, pad_idx % N]).reshape(16, NB, B)
    dst_p = jnp.concatenate([dst, N + pad_idx % (N_PAD - N)]).reshape(16, NB, B)

    x_pad = jnp.concatenate([x, jnp.zeros((N_PAD - N, 128), jnp.float32)])

    w2s = W2.T.reshape(128, 2, HALF).transpose(1, 0, 2)
    b2s = b2.reshape(2, 1, HALF)
    h2 = _mlp(x_pad, W1.T, b1.reshape(1, -1), w2s, b2s)
    h_flat = h2.reshape(2 * N_PAD, HALF)

    outp = _appnp_sc(h_flat, src_p, dst_p)
    return jnp.concatenate([outp[0, :N], outp[1, :N]], axis=1)


# speedup vs baseline: 1.2590x; 1.2590x over previous
import functools

import jax
import jax.numpy as jnp
from jax import lax
from jax.experimental import pallas as pl
from jax.experimental.pallas import tpu as pltpu
from jax.experimental.pallas import tpu_sc as plsc

N = 10000
N_PAD = 10240
E = 320000
E2 = E + N
HALF = 32
K = 5
ALPHA = 0.1
B = 128
NBUF = 8
NB = 168
CB = 56
NCB = NB // CB
CBT = CB // NBUF
E_PAD = 16 * NB * B
RPT = N_PAD // 16

_mesh = plsc.VectorSubcoreMesh(core_axis_name="c", subcore_axis_name="s")


def _mlp_body(x_ref, w1_ref, b1_ref, w2_ref, b2_ref, o_ref):
    h = jnp.maximum(
        jnp.dot(x_ref[...], w1_ref[...], preferred_element_type=jnp.float32)
        + b1_ref[...],
        0.0,
    )
    o_ref[0] = (
        jnp.dot(h, w2_ref[0], preferred_element_type=jnp.float32) + b2_ref[0]
    )


def _mlp(x_pad, w1t, b1, w2t, b2):
    blk = 1024
    return pl.pallas_call(
        _mlp_body,
        grid=(2, N_PAD // blk),
        in_specs=[
            pl.BlockSpec((blk, 128), lambda c, i: (i, 0)),
            pl.BlockSpec((128, 128), lambda c, i: (0, 0)),
            pl.BlockSpec((1, 128), lambda c, i: (0, 0)),
            pl.BlockSpec((1, 128, HALF), lambda c, i: (c, 0, 0)),
            pl.BlockSpec((1, 1, HALF), lambda c, i: (c, 0, 0)),
        ],
        out_specs=pl.BlockSpec((1, blk, HALF), lambda c, i: (c, i, 0)),
        out_shape=jax.ShapeDtypeStruct((2, N_PAD, HALF), jnp.float32),
    )(x_pad, w1t, b1, w2t, b2)


def _rsqrt16(x):
    i = lax.bitcast_convert_type(x, jnp.int32)
    i = jnp.int32(0x5F3759DF) - (i >> 1)
    y = lax.bitcast_convert_type(i, jnp.float32)
    for _ in range(4):
        y = y * (1.5 - 0.5 * x * y * y)
    return y


def _appnp_body(h_hbm, src_hbm, dst_hbm, u_hbm, out_hbm,
                src_v, dst_v, b0, b1, b2, b3, b4, b5, b6, b7,
                ones_v, dinv_v, c2_v, strip_v, s_sp,
                m0, m1, m2, m3, m4, m5, m6, m7):
    bufs = (b0, b1, b2, b3, b4, b5, b6, b7)
    sems = (m0, m1, m2, m3, m4, m5, m6, m7)
    c = lax.axis_index("c")
    s = lax.axis_index("s")
    r0 = s * RPT
    ubase = c * N_PAD + r0

    one16 = jnp.full((16,), 1.0, jnp.float32)
    zero16 = jnp.zeros((16,), jnp.float32)

    def fill_ones(j, carry):
        ones_v[j, pl.ds(0, 16)] = one16
        ones_v[j, pl.ds(16, 16)] = one16
        return carry

    lax.fori_loop(0, B, fill_ones, 0, unroll=False)

    def zero_strip(r, carry):
        strip_v[r, pl.ds(0, 16)] = zero16
        strip_v[r, pl.ds(16, 16)] = zero16
        return carry

    lax.fori_loop(0, RPT, zero_strip, 0, unroll=False)

    pltpu.sync_copy(strip_v, s_sp.at[pl.ds(r0, RPT)])
    plsc.subcore_barrier()

    def degchunk(q, carry):
        pltpu.sync_copy(dst_hbm.at[s, pl.ds(q * CB, CB)], dst_v)

        def degblk(j, cr):
            pltpu.sync_copy(ones_v, s_sp.at[dst_v.at[j]], add=True)
            return cr

        lax.fori_loop(0, CB, degblk, 0, unroll=False)
        return carry

    lax.fori_loop(0, NCB, degchunk, 0, unroll=False)
    plsc.subcore_barrier()

    pltpu.sync_copy(s_sp.at[pl.ds(r0, RPT)], strip_v)

    def zero_c2(r, carry):
        c2_v[r, pl.ds(0, 16)] = zero16
        c2_v[r, pl.ds(16, 16)] = zero16
        return carry

    lax.fori_loop(0, RPT, zero_c2, 0, unroll=False)
    pltpu.sync_copy(c2_v, s_sp.at[pl.ds(r0, RPT)])
    pltpu.sync_copy(h_hbm.at[pl.ds(ubase, RPT)], c2_v)

    def setup_row(r, carry):
        d = _rsqrt16(strip_v[r, pl.ds(0, 16)])
        dinv_v[r] = d
        h0 = c2_v[r, pl.ds(0, 16)]
        h1 = c2_v[r, pl.ds(16, 16)]
        c2_v[r, pl.ds(0, 16)] = ALPHA * h0
        c2_v[r, pl.ds(16, 16)] = ALPHA * h1
        strip_v[r, pl.ds(0, 16)] = d * h0
        strip_v[r, pl.ds(16, 16)] = d * h1
        return carry

    lax.fori_loop(0, RPT, setup_row, 0, unroll=False)
    pltpu.sync_copy(strip_v, u_hbm.at[pl.ds(ubase, RPT)])
    plsc.subcore_barrier()

    def hop(k, carry):
        def chunk(q, cq):
            pltpu.sync_copy(src_hbm.at[c, s, pl.ds(q * CB, CB)], src_v)
            pltpu.sync_copy(dst_hbm.at[s, pl.ds(q * CB, CB)], dst_v)
            for b in range(NBUF):
                pltpu.async_copy(u_hbm.at[src_v.at[b]], bufs[b], sems[b])

            def blk(t, cr):
                base = t * NBUF
                for b in range(NBUF):
                    j = base + b
                    pltpu.make_async_copy(u_hbm.at[src_v.at[j]], bufs[b],
                                          sems[b]).wait()
                    pltpu.sync_copy(bufs[b], s_sp.at[dst_v.at[j]], add=True)
                    pltpu.async_copy(u_hbm.at[src_v.at[j + NBUF]], bufs[b],
                                     sems[b])
                return cr

            lax.fori_loop(0, CBT - 1, blk, 0, unroll=False)
            base = (CBT - 1) * NBUF
            for b in range(NBUF):
                j = base + b
                pltpu.make_async_copy(u_hbm.at[src_v.at[j]], bufs[b],
                                      sems[b]).wait()
                pltpu.sync_copy(bufs[b], s_sp.at[dst_v.at[j]], add=True)
            return cq

        lax.fori_loop(0, NCB, chunk, 0, unroll=False)
        plsc.subcore_barrier()

        pltpu.sync_copy(s_sp.at[pl.ds(r0, RPT)], strip_v)
        last = k == K - 1

        def comb(r, cr):
            d = dinv_v[r]
            s0 = strip_v[r, pl.ds(0, 16)]
            s1 = strip_v[r, pl.ds(16, 16)]
            e0 = c2_v[r, pl.ds(0, 16)]
            e1 = c2_v[r, pl.ds(16, 16)]
            ds0 = (1.0 - ALPHA) * d * s0
            ds1 = (1.0 - ALPHA) * d * s1
            strip_v[r, pl.ds(0, 16)] = jnp.where(last, ds0 + e0,
                                                 d * (ds0 + e0))
            strip_v[r, pl.ds(16, 16)] = jnp.where(last, ds1 + e1,
                                                  d * (ds1 + e1))
            return cr

        lax.fori_loop(0, RPT, comb, 0, unroll=False)

        pltpu.sync_copy(strip_v, u_hbm.at[pl.ds(ubase, RPT)])

        @pl.when(last)
        def _():
            pltpu.sync_copy(strip_v, out_hbm.at[c, pl.ds(r0, RPT)])

        lax.fori_loop(0, RPT, zero_strip, 0, unroll=False)
        pltpu.sync_copy(strip_v, s_sp.at[pl.ds(r0, RPT)])
        plsc.subcore_barrier()
        return carry

    lax.fori_loop(0, K, hop, 0, unroll=False)


@functools.partial(
    pl.kernel,
    out_type=[
        jax.ShapeDtypeStruct((2 * N_PAD, HALF), jnp.float32),
        jax.ShapeDtypeStruct((2, N_PAD, HALF), jnp.float32),
    ],
    mesh=_mesh,
    compiler_params=pltpu.CompilerParams(use_tc_tiling_on_sc=False),
    scratch_types=[
        pltpu.VMEM((CB, B), jnp.int32),
        pltpu.VMEM((CB, B), jnp.int32),
        pltpu.VMEM((B, HALF), jnp.float32),
        pltpu.VMEM((B, HALF), jnp.float32),
        pltpu.VMEM((B, HALF), jnp.float32),
        pltpu.VMEM((B, HALF), jnp.float32),
        pltpu.VMEM((B, HALF), jnp.float32),
        pltpu.VMEM((B, HALF), jnp.float32),
        pltpu.VMEM((B, HALF), jnp.float32),
        pltpu.VMEM((B, HALF), jnp.float32),
        pltpu.VMEM((B, HALF), jnp.float32),
        pltpu.VMEM((RPT, 16), jnp.float32),
        pltpu.VMEM((RPT, HALF), jnp.float32),
        pltpu.VMEM((RPT, HALF), jnp.float32),
        pltpu.VMEM_SHARED((N_PAD, HALF), jnp.float32),
        pltpu.SemaphoreType.DMA,
        pltpu.SemaphoreType.DMA,
        pltpu.SemaphoreType.DMA,
        pltpu.SemaphoreType.DMA,
        pltpu.SemaphoreType.DMA,
        pltpu.SemaphoreType.DMA,
        pltpu.SemaphoreType.DMA,
        pltpu.SemaphoreType.DMA,
    ],
)
def _appnp_sc(h_hbm, src_hbm, dst_hbm, u_hbm, out_hbm,
              src_v, dst_v, b0, b1, b2, b3, b4, b5, b6, b7,
              ones_v, dinv_v, c2_v, strip_v, s_sp,
              m0, m1, m2, m3, m4, m5, m6, m7):
    _appnp_body(h_hbm, src_hbm, dst_hbm, u_hbm, out_hbm,
                src_v, dst_v, b0, b1, b2, b3, b4, b5, b6, b7,
                ones_v, dinv_v, c2_v, strip_v, s_sp,
                m0, m1, m2, m3, m4, m5, m6, m7)


def kernel(x, edge_index, W1, b1, W2, b2):
    loop = jnp.arange(N, dtype=jnp.int32)
    src = jnp.concatenate([edge_index[0], loop])
    dst = jnp.concatenate([edge_index[1], loop])
    pad = E_PAD - E2
    pad_idx = jnp.arange(pad, dtype=jnp.int32)
    src_p = jnp.concatenate([src, pad_idx % N]).reshape(16, NB, B)
    dst_p = jnp.concatenate([dst, N + pad_idx % (N_PAD - N)]).reshape(16, NB, B)
    src_b = jnp.stack([src_p, src_p + N_PAD])

    x_pad = jnp.concatenate([x, jnp.zeros((N_PAD - N, 128), jnp.float32)])

    w2s = W2.T.reshape(128, 2, HALF).transpose(1, 0, 2)
    b2s = b2.reshape(2, 1, HALF)
    h2 = _mlp(x_pad, W1.T, b1.reshape(1, -1), w2s, b2s)
    h_flat = h2.reshape(2 * N_PAD, HALF)

    _, outp = _appnp_sc(h_flat, src_b, dst_p)
    return jnp.concatenate([outp[0, :N], outp[1, :N]], axis=1)


# speedup vs baseline: 1.3296x; 1.0560x over previous
import functools

import jax
import jax.numpy as jnp
from jax import lax
from jax.experimental import pallas as pl
from jax.experimental.pallas import tpu as pltpu
from jax.experimental.pallas import tpu_sc as plsc

N = 10000
N_PAD = 10240
E = 320000
HALF = 32
K = 5
ALPHA = 0.1
B = 125
NBUF = 8
NB = 160
CB = 40
NCB = NB // CB
CBT = CB // NBUF
RPT = N_PAD // 16

_mesh = plsc.VectorSubcoreMesh(core_axis_name="c", subcore_axis_name="s")


def _mlp_body(x_ref, w1_ref, b1_ref, w2_ref, b2_ref, o_ref):
    h = jnp.maximum(
        jnp.dot(x_ref[...], w1_ref[...], preferred_element_type=jnp.float32)
        + b1_ref[...],
        0.0,
    )
    o_ref[0] = (
        jnp.dot(h, w2_ref[0], preferred_element_type=jnp.float32) + b2_ref[0]
    )


def _mlp(x_pad, w1t, b1, w2t, b2):
    blk = 1024
    return pl.pallas_call(
        _mlp_body,
        grid=(2, N_PAD // blk),
        in_specs=[
            pl.BlockSpec((blk, 128), lambda c, i: (i, 0)),
            pl.BlockSpec((128, 128), lambda c, i: (0, 0)),
            pl.BlockSpec((1, 128), lambda c, i: (0, 0)),
            pl.BlockSpec((1, 128, HALF), lambda c, i: (c, 0, 0)),
            pl.BlockSpec((1, 1, HALF), lambda c, i: (c, 0, 0)),
        ],
        out_specs=pl.BlockSpec((1, blk, HALF), lambda c, i: (c, i, 0)),
        out_shape=jax.ShapeDtypeStruct((2, N_PAD, HALF), jnp.float32),
    )(x_pad, w1t, b1, w2t, b2)


def _rsqrt16(x):
    i = lax.bitcast_convert_type(x, jnp.int32)
    i = jnp.int32(0x5F3759DF) - (i >> 1)
    y = lax.bitcast_convert_type(i, jnp.float32)
    for _ in range(4):
        y = y * (1.5 - 0.5 * x * y * y)
    return y


def _appnp_body(h_hbm, src_hbm, dst_hbm, u_hbm, out_hbm,
                src_v, dst_v, b0, b1, b2, b3, b4, b5, b6, b7,
                ones_v, dinv_v, c2_v, strip_v, s_sp,
                m0, m1, m2, m3, m4, m5, m6, m7):
    bufs = (b0, b1, b2, b3, b4, b5, b6, b7)
    sems = (m0, m1, m2, m3, m4, m5, m6, m7)
    c = lax.axis_index("c")
    s = lax.axis_index("s")
    r0 = s * RPT
    ubase = c * N_PAD + r0

    one16 = jnp.full((16,), 1.0, jnp.float32)
    zero16 = jnp.zeros((16,), jnp.float32)

    def fill_ones(j, carry):
        ones_v[j, pl.ds(0, 16)] = one16
        ones_v[j, pl.ds(16, 16)] = one16
        return carry

    lax.fori_loop(0, B, fill_ones, 0, unroll=False)

    def zero_strip(r, carry):
        strip_v[r, pl.ds(0, 16)] = zero16
        strip_v[r, pl.ds(16, 16)] = zero16
        return carry

    lax.fori_loop(0, RPT, zero_strip, 0, unroll=False)

    pltpu.sync_copy(strip_v, s_sp.at[pl.ds(r0, RPT)])
    plsc.subcore_barrier()

    def degchunk(q, carry):
        pltpu.sync_copy(dst_hbm.at[s, pl.ds(q * CB, CB)], dst_v)

        def degblk(j, cr):
            pltpu.sync_copy(ones_v, s_sp.at[dst_v.at[j]], add=True)
            return cr

        lax.fori_loop(0, CB, degblk, 0, unroll=False)
        return carry

    lax.fori_loop(0, NCB, degchunk, 0, unroll=False)
    plsc.subcore_barrier()

    pltpu.sync_copy(s_sp.at[pl.ds(r0, RPT)], strip_v)

    pltpu.sync_copy(h_hbm.at[pl.ds(ubase, RPT)], c2_v)

    def setup_row(r, carry):
        d = _rsqrt16(strip_v[r, pl.ds(0, 16)] + 1.0)
        dinv_v[r] = d
        h0 = c2_v[r, pl.ds(0, 16)]
        h1 = c2_v[r, pl.ds(16, 16)]
        c2_v[r, pl.ds(0, 16)] = ALPHA * h0
        c2_v[r, pl.ds(16, 16)] = ALPHA * h1
        strip_v[r, pl.ds(0, 16)] = d * h0
        strip_v[r, pl.ds(16, 16)] = d * h1
        return carry

    lax.fori_loop(0, RPT, setup_row, 0, unroll=False)
    pltpu.sync_copy(strip_v, u_hbm.at[pl.ds(ubase, RPT)])
    pltpu.sync_copy(strip_v, s_sp.at[pl.ds(r0, RPT)])
    plsc.subcore_barrier()

    def hop(k, carry):
        def chunk(q, cq):
            pltpu.sync_copy(src_hbm.at[c, s, pl.ds(q * CB, CB)], src_v)
            pltpu.sync_copy(dst_hbm.at[s, pl.ds(q * CB, CB)], dst_v)
            for b in range(NBUF):
                pltpu.async_copy(u_hbm.at[src_v.at[b]], bufs[b], sems[b])

            def blk(t, cr):
                base = t * NBUF
                for b in range(NBUF):
                    j = base + b
                    pltpu.make_async_copy(u_hbm.at[src_v.at[j]], bufs[b],
                                          sems[b]).wait()
                    pltpu.sync_copy(bufs[b], s_sp.at[dst_v.at[j]], add=True)
                    pltpu.async_copy(u_hbm.at[src_v.at[j + NBUF]], bufs[b],
                                     sems[b])
                return cr

            lax.fori_loop(0, CBT - 1, blk, 0, unroll=False)
            base = (CBT - 1) * NBUF
            for b in range(NBUF):
                j = base + b
                pltpu.make_async_copy(u_hbm.at[src_v.at[j]], bufs[b],
                                      sems[b]).wait()
                pltpu.sync_copy(bufs[b], s_sp.at[dst_v.at[j]], add=True)
            return cq

        lax.fori_loop(0, NCB, chunk, 0, unroll=False)
        plsc.subcore_barrier()

        pltpu.sync_copy(s_sp.at[pl.ds(r0, RPT)], strip_v)
        last = k == K - 1

        def comb(r, cr):
            d = dinv_v[r]
            s0 = strip_v[r, pl.ds(0, 16)]
            s1 = strip_v[r, pl.ds(16, 16)]
            e0 = c2_v[r, pl.ds(0, 16)]
            e1 = c2_v[r, pl.ds(16, 16)]
            ds0 = (1.0 - ALPHA) * d * s0
            ds1 = (1.0 - ALPHA) * d * s1
            strip_v[r, pl.ds(0, 16)] = jnp.where(last, ds0 + e0,
                                                 d * (ds0 + e0))
            strip_v[r, pl.ds(16, 16)] = jnp.where(last, ds1 + e1,
                                                  d * (ds1 + e1))
            return cr

        lax.fori_loop(0, RPT, comb, 0, unroll=False)

        pltpu.sync_copy(strip_v, u_hbm.at[pl.ds(ubase, RPT)])

        @pl.when(last)
        def _():
            pltpu.sync_copy(strip_v, out_hbm.at[c, pl.ds(r0, RPT)])

        pltpu.sync_copy(strip_v, s_sp.at[pl.ds(r0, RPT)])
        plsc.subcore_barrier()
        return carry

    lax.fori_loop(0, K, hop, 0, unroll=False)


@functools.partial(
    pl.kernel,
    out_type=[
        jax.ShapeDtypeStruct((2 * N_PAD, HALF), jnp.float32),
        jax.ShapeDtypeStruct((2, N_PAD, HALF), jnp.float32),
    ],
    mesh=_mesh,
    compiler_params=pltpu.CompilerParams(use_tc_tiling_on_sc=False),
    scratch_types=[
        pltpu.VMEM((CB, B), jnp.int32),
        pltpu.VMEM((CB, B), jnp.int32),
        pltpu.VMEM((B, HALF), jnp.float32),
        pltpu.VMEM((B, HALF), jnp.float32),
        pltpu.VMEM((B, HALF), jnp.float32),
        pltpu.VMEM((B, HALF), jnp.float32),
        pltpu.VMEM((B, HALF), jnp.float32),
        pltpu.VMEM((B, HALF), jnp.float32),
        pltpu.VMEM((B, HALF), jnp.float32),
        pltpu.VMEM((B, HALF), jnp.float32),
        pltpu.VMEM((B, HALF), jnp.float32),
        pltpu.VMEM((RPT, 16), jnp.float32),
        pltpu.VMEM((RPT, HALF), jnp.float32),
        pltpu.VMEM((RPT, HALF), jnp.float32),
        pltpu.VMEM_SHARED((N_PAD, HALF), jnp.float32),
        pltpu.SemaphoreType.DMA,
        pltpu.SemaphoreType.DMA,
        pltpu.SemaphoreType.DMA,
        pltpu.SemaphoreType.DMA,
        pltpu.SemaphoreType.DMA,
        pltpu.SemaphoreType.DMA,
        pltpu.SemaphoreType.DMA,
        pltpu.SemaphoreType.DMA,
    ],
)
def _appnp_sc(h_hbm, src_hbm, dst_hbm, u_hbm, out_hbm,
              src_v, dst_v, b0, b1, b2, b3, b4, b5, b6, b7,
              ones_v, dinv_v, c2_v, strip_v, s_sp,
              m0, m1, m2, m3, m4, m5, m6, m7):
    _appnp_body(h_hbm, src_hbm, dst_hbm, u_hbm, out_hbm,
                src_v, dst_v, b0, b1, b2, b3, b4, b5, b6, b7,
                ones_v, dinv_v, c2_v, strip_v, s_sp,
                m0, m1, m2, m3, m4, m5, m6, m7)


def kernel(x, edge_index, W1, b1, W2, b2):
    src_p = edge_index[0].reshape(16, NB, B)
    dst_p = edge_index[1].reshape(16, NB, B)
    src_b = jnp.stack([src_p, src_p + N_PAD])

    x_pad = jnp.concatenate([x, jnp.zeros((N_PAD - N, 128), jnp.float32)])

    w2s = W2.T.reshape(128, 2, HALF).transpose(1, 0, 2)
    b2s = b2.reshape(2, 1, HALF)
    h2 = _mlp(x_pad, W1.T, b1.reshape(1, -1), w2s, b2s)
    h_flat = h2.reshape(2 * N_PAD, HALF)

    _, outp = _appnp_sc(h_flat, src_b, dst_p)
    return jnp.concatenate([outp[0, :N], outp[1, :N]], axis=1)


# speedup vs baseline: 1.3563x; 1.0201x over previous
import functools

import jax
import jax.numpy as jnp
from jax import lax
from jax.experimental import pallas as pl
from jax.experimental.pallas import tpu as pltpu
from jax.experimental.pallas import tpu_sc as plsc

N = 10000
N_PAD = 10240
E = 320000
HALF = 32
K = 5
ALPHA = 0.1
B = 125
NBUF = 8
NB = 160
CB = 40
NCB = NB // CB
CBT = CB // NBUF
RPT = N_PAD // 16

_mesh = plsc.VectorSubcoreMesh(core_axis_name="c", subcore_axis_name="s")


def _mlp_body(x_ref, w1_ref, b1_ref, w2_ref, b2_ref, o_ref):
    dn = (((1,), (1,)), ((), ()))
    h = jnp.maximum(
        lax.dot_general(x_ref[...], w1_ref[...], dn,
                        preferred_element_type=jnp.float32) + b1_ref[...],
        0.0,
    )
    o_ref[0] = (
        lax.dot_general(h, w2_ref[0], dn,
                        preferred_element_type=jnp.float32) + b2_ref[0]
    )


def _mlp(x_pad, w1t, b1, w2t, b2):
    blk = 1024
    return pl.pallas_call(
        _mlp_body,
        grid=(2, N_PAD // blk),
        in_specs=[
            pl.BlockSpec((blk, 128), lambda c, i: (i, 0)),
            pl.BlockSpec((128, 128), lambda c, i: (0, 0)),
            pl.BlockSpec((1, 128), lambda c, i: (0, 0)),
            pl.BlockSpec((1, HALF, 128), lambda c, i: (c, 0, 0)),
            pl.BlockSpec((1, 1, HALF), lambda c, i: (c, 0, 0)),
        ],
        out_specs=pl.BlockSpec((1, blk, HALF), lambda c, i: (c, i, 0)),
        out_shape=jax.ShapeDtypeStruct((2, N_PAD, HALF), jnp.float32),
    )(x_pad, w1t, b1, w2t, b2)


def _rsqrt16(x):
    i = lax.bitcast_convert_type(x, jnp.int32)
    i = jnp.int32(0x5F3759DF) - (i >> 1)
    y = lax.bitcast_convert_type(i, jnp.float32)
    for _ in range(4):
        y = y * (1.5 - 0.5 * x * y * y)
    return y


def _appnp_body(h_hbm, src_hbm, dst_hbm, u_hbm, out_hbm,
                src_v, dst_v, b0, b1, b2, b3, b4, b5, b6, b7,
                ones_v, dinv_v, c2_v, strip_v, s_sp,
                m0, m1, m2, m3, m4, m5, m6, m7):
    bufs = (b0, b1, b2, b3, b4, b5, b6, b7)
    sems = (m0, m1, m2, m3, m4, m5, m6, m7)
    c = lax.axis_index("c")
    s = lax.axis_index("s")
    r0 = s * RPT
    ubase = c * N_PAD + r0

    one16 = jnp.full((16,), 1.0, jnp.float32)
    zero16 = jnp.zeros((16,), jnp.float32)

    def fill_ones(j, carry):
        ones_v[j, pl.ds(0, 16)] = one16
        ones_v[j, pl.ds(16, 16)] = one16
        return carry

    lax.fori_loop(0, B, fill_ones, 0, unroll=False)

    def zero_strip(r, carry):
        strip_v[r, pl.ds(0, 16)] = zero16
        strip_v[r, pl.ds(16, 16)] = zero16
        return carry

    lax.fori_loop(0, RPT, zero_strip, 0, unroll=False)

    pltpu.sync_copy(strip_v, s_sp.at[pl.ds(r0, RPT)])
    plsc.subcore_barrier()

    def degchunk(q, carry):
        pltpu.sync_copy(dst_hbm.at[s, pl.ds(q * CB, CB)], dst_v)

        def degblk(j, cr):
            pltpu.sync_copy(ones_v, s_sp.at[dst_v.at[j]], add=True)
            return cr

        lax.fori_loop(0, CB, degblk, 0, unroll=False)
        return carry

    lax.fori_loop(0, NCB, degchunk, 0, unroll=False)
    plsc.subcore_barrier()

    pltpu.sync_copy(s_sp.at[pl.ds(r0, RPT)], strip_v)

    pltpu.sync_copy(h_hbm.at[c, pl.ds(r0, RPT)], c2_v)

    def setup_row(r, carry):
        d = _rsqrt16(strip_v[r, pl.ds(0, 16)] + 1.0)
        dinv_v[r] = d
        h0 = c2_v[r, pl.ds(0, 16)]
        h1 = c2_v[r, pl.ds(16, 16)]
        c2_v[r, pl.ds(0, 16)] = ALPHA * h0
        c2_v[r, pl.ds(16, 16)] = ALPHA * h1
        strip_v[r, pl.ds(0, 16)] = d * h0
        strip_v[r, pl.ds(16, 16)] = d * h1
        return carry

    lax.fori_loop(0, RPT, setup_row, 0, unroll=False)
    pltpu.sync_copy(strip_v, u_hbm.at[pl.ds(ubase, RPT)])
    pltpu.sync_copy(strip_v, s_sp.at[pl.ds(r0, RPT)])
    plsc.subcore_barrier()

    def hop(k, carry):
        def chunk(q, cq):
            pltpu.sync_copy(src_hbm.at[c, s, pl.ds(q * CB, CB)], src_v)
            pltpu.sync_copy(dst_hbm.at[s, pl.ds(q * CB, CB)], dst_v)
            for b in range(NBUF):
                pltpu.async_copy(u_hbm.at[src_v.at[b]], bufs[b], sems[b])

            def blk(t, cr):
                base = t * NBUF
                for b in range(NBUF):
                    j = base + b
                    pltpu.make_async_copy(u_hbm.at[src_v.at[j]], bufs[b],
                                          sems[b]).wait()
                    pltpu.sync_copy(bufs[b], s_sp.at[dst_v.at[j]], add=True)
                    pltpu.async_copy(u_hbm.at[src_v.at[j + NBUF]], bufs[b],
                                     sems[b])
                return cr

            lax.fori_loop(0, CBT - 1, blk, 0, unroll=False)
            base = (CBT - 1) * NBUF
            for b in range(NBUF):
                j = base + b
                pltpu.make_async_copy(u_hbm.at[src_v.at[j]], bufs[b],
                                      sems[b]).wait()
                pltpu.sync_copy(bufs[b], s_sp.at[dst_v.at[j]], add=True)
            return cq

        lax.fori_loop(0, NCB, chunk, 0, unroll=False)
        plsc.subcore_barrier()

        pltpu.sync_copy(s_sp.at[pl.ds(r0, RPT)], strip_v)
        last = k == K - 1

        def comb(r, cr):
            d = dinv_v[r]
            s0 = strip_v[r, pl.ds(0, 16)]
            s1 = strip_v[r, pl.ds(16, 16)]
            e0 = c2_v[r, pl.ds(0, 16)]
            e1 = c2_v[r, pl.ds(16, 16)]
            ds0 = (1.0 - ALPHA) * d * s0
            ds1 = (1.0 - ALPHA) * d * s1
            strip_v[r, pl.ds(0, 16)] = jnp.where(last, ds0 + e0,
                                                 d * (ds0 + e0))
            strip_v[r, pl.ds(16, 16)] = jnp.where(last, ds1 + e1,
                                                  d * (ds1 + e1))
            return cr

        lax.fori_loop(0, RPT, comb, 0, unroll=False)

        pltpu.sync_copy(strip_v, u_hbm.at[pl.ds(ubase, RPT)])

        @pl.when(last)
        def _():
            pltpu.sync_copy(strip_v,
                            out_hbm.at[pl.ds(r0, RPT), pl.ds(c * HALF, HALF)])

        pltpu.sync_copy(strip_v, s_sp.at[pl.ds(r0, RPT)])
        plsc.subcore_barrier()
        return carry

    lax.fori_loop(0, K, hop, 0, unroll=False)


@functools.partial(
    pl.kernel,
    out_type=[
        jax.ShapeDtypeStruct((2 * N_PAD, HALF), jnp.float32),
        jax.ShapeDtypeStruct((N_PAD, 2 * HALF), jnp.float32),
    ],
    mesh=_mesh,
    compiler_params=pltpu.CompilerParams(use_tc_tiling_on_sc=False),
    scratch_types=[
        pltpu.VMEM((CB, B), jnp.int32),
        pltpu.VMEM((CB, B), jnp.int32),
        pltpu.VMEM((B, HALF), jnp.float32),
        pltpu.VMEM((B, HALF), jnp.float32),
        pltpu.VMEM((B, HALF), jnp.float32),
        pltpu.VMEM((B, HALF), jnp.float32),
        pltpu.VMEM((B, HALF), jnp.float32),
        pltpu.VMEM((B, HALF), jnp.float32),
        pltpu.VMEM((B, HALF), jnp.float32),
        pltpu.VMEM((B, HALF), jnp.float32),
        pltpu.VMEM((B, HALF), jnp.float32),
        pltpu.VMEM((RPT, 16), jnp.float32),
        pltpu.VMEM((RPT, HALF), jnp.float32),
        pltpu.VMEM((RPT, HALF), jnp.float32),
        pltpu.VMEM_SHARED((N_PAD, HALF), jnp.float32),
        pltpu.SemaphoreType.DMA,
        pltpu.SemaphoreType.DMA,
        pltpu.SemaphoreType.DMA,
        pltpu.SemaphoreType.DMA,
        pltpu.SemaphoreType.DMA,
        pltpu.SemaphoreType.DMA,
        pltpu.SemaphoreType.DMA,
        pltpu.SemaphoreType.DMA,
    ],
)
def _appnp_sc(h_hbm, src_hbm, dst_hbm, u_hbm, out_hbm,
              src_v, dst_v, b0, b1, b2, b3, b4, b5, b6, b7,
              ones_v, dinv_v, c2_v, strip_v, s_sp,
              m0, m1, m2, m3, m4, m5, m6, m7):
    _appnp_body(h_hbm, src_hbm, dst_hbm, u_hbm, out_hbm,
                src_v, dst_v, b0, b1, b2, b3, b4, b5, b6, b7,
                ones_v, dinv_v, c2_v, strip_v, s_sp,
                m0, m1, m2, m3, m4, m5, m6, m7)


def kernel(x, edge_index, W1, b1, W2, b2):
    src_p = edge_index[0].reshape(16, NB, B)
    dst_p = edge_index[1].reshape(16, NB, B)
    src_b = jnp.stack([src_p, src_p + N_PAD])

    x_pad = jnp.concatenate([x, jnp.zeros((N_PAD - N, 128), jnp.float32)])

    w2s = W2.reshape(2, HALF, 128)
    b2s = b2.reshape(2, 1, HALF)
    h2 = _mlp(x_pad, W1, b1.reshape(1, -1), w2s, b2s)

    _, outp = _appnp_sc(h2, src_b, dst_p)
    return outp[:N]


# speedup vs baseline: 1.4424x; 1.0634x over previous
import functools

import jax
import jax.numpy as jnp
from jax import lax
from jax.experimental import pallas as pl
from jax.experimental.pallas import tpu as pltpu
from jax.experimental.pallas import tpu_sc as plsc

N = 10000
N_PAD = 10240
E = 320000
HALF = 32
K = 5
ALPHA = 0.1
B = 125
NBUF = 8
NB = 160
CB = 40
NCB = NB // CB
CBT = CB // NBUF
RPT = N_PAD // 16

_mesh = plsc.VectorSubcoreMesh(core_axis_name="c", subcore_axis_name="s")


def _mlp_body(x_ref, w1_ref, b1_ref, w2_ref, b2_ref, o_ref):
    dn = (((1,), (1,)), ((), ()))
    h = jnp.maximum(
        lax.dot_general(x_ref[...], w1_ref[...], dn,
                        preferred_element_type=jnp.float32) + b1_ref[...],
        0.0,
    )
    o_ref[0] = (
        lax.dot_general(h, w2_ref[0], dn,
                        preferred_element_type=jnp.float32) + b2_ref[0]
    )


def _mlp(x_pad, w1t, b1, w2t, b2):
    blk = 1024
    return pl.pallas_call(
        _mlp_body,
        grid=(2, N_PAD // blk),
        in_specs=[
            pl.BlockSpec((blk, 128), lambda c, i: (i, 0)),
            pl.BlockSpec((128, 128), lambda c, i: (0, 0)),
            pl.BlockSpec((1, 128), lambda c, i: (0, 0)),
            pl.BlockSpec((1, HALF, 128), lambda c, i: (c, 0, 0)),
            pl.BlockSpec((1, 1, HALF), lambda c, i: (c, 0, 0)),
        ],
        out_specs=pl.BlockSpec((1, blk, HALF), lambda c, i: (c, i, 0)),
        out_shape=jax.ShapeDtypeStruct((2, N_PAD, HALF), jnp.float32),
    )(x_pad, w1t, b1, w2t, b2)


def _rsqrt16(x):
    i = lax.bitcast_convert_type(x, jnp.int32)
    i = jnp.int32(0x5F3759DF) - (i >> 1)
    y = lax.bitcast_convert_type(i, jnp.float32)
    for _ in range(4):
        y = y * (1.5 - 0.5 * x * y * y)
    return y


def _appnp_body(h_hbm, src_hbm, dst_hbm, u_hbm, out_hbm,
                src_va, dst_va, src_vb, dst_vb,
                b0, b1, b2, b3, b4, b5, b6, b7,
                ones_v, dinv_v, c2_v, strip_v, s_sp,
                m0, m1, m2, m3, m4, m5, m6, m7, mia, mib):
    bufs = (b0, b1, b2, b3, b4, b5, b6, b7)
    sems = (m0, m1, m2, m3, m4, m5, m6, m7)
    c = lax.axis_index("c")
    s = lax.axis_index("s")
    r0 = s * RPT
    ubase = c * N_PAD + r0

    one16 = jnp.full((16,), 1.0, jnp.float32)
    zero16 = jnp.zeros((16,), jnp.float32)

    def fill_ones(j, carry):
        ones_v[j, pl.ds(0, 16)] = one16
        ones_v[j, pl.ds(16, 16)] = one16
        return carry

    lax.fori_loop(0, B, fill_ones, 0, unroll=False)

    def zero_strip(r, carry):
        strip_v[r, pl.ds(0, 16)] = zero16
        strip_v[r, pl.ds(16, 16)] = zero16
        return carry

    lax.fori_loop(0, RPT, zero_strip, 0, unroll=False)

    pltpu.sync_copy(strip_v, s_sp.at[pl.ds(r0, RPT)])
    plsc.subcore_barrier()

    def degchunk(q, carry):
        pltpu.sync_copy(dst_hbm.at[s, pl.ds(q * CB, CB)], dst_va)

        def degblk(j, cr):
            pltpu.sync_copy(ones_v, s_sp.at[dst_va.at[j]], add=True)
            return cr

        lax.fori_loop(0, CB, degblk, 0, unroll=False)
        return carry

    lax.fori_loop(0, NCB, degchunk, 0, unroll=False)
    plsc.subcore_barrier()

    pltpu.sync_copy(s_sp.at[pl.ds(r0, RPT)], strip_v)

    pltpu.sync_copy(h_hbm.at[c, pl.ds(r0, RPT)], c2_v)

    def setup_row(r, carry):
        d = _rsqrt16(strip_v[r, pl.ds(0, 16)] + 1.0)
        dinv_v[r] = d
        h0 = c2_v[r, pl.ds(0, 16)]
        h1 = c2_v[r, pl.ds(16, 16)]
        c2_v[r, pl.ds(0, 16)] = ALPHA * h0
        c2_v[r, pl.ds(16, 16)] = ALPHA * h1
        strip_v[r, pl.ds(0, 16)] = d * h0
        strip_v[r, pl.ds(16, 16)] = d * h1
        return carry

    lax.fori_loop(0, RPT, setup_row, 0, unroll=False)
    pltpu.sync_copy(strip_v, u_hbm.at[pl.ds(ubase, RPT)])
    pltpu.sync_copy(strip_v, s_sp.at[pl.ds(r0, RPT)])
    plsc.subcore_barrier()

    NCB2 = NCB // 2

    def _proc(src_v, dst_v):
        for b in range(NBUF):
            pltpu.async_copy(u_hbm.at[src_v.at[b]], bufs[b], sems[b])

        def blk(t, cr):
            base = t * NBUF
            for b in range(NBUF):
                j = base + b
                pltpu.make_async_copy(u_hbm.at[src_v.at[j]], bufs[b],
                                      sems[b]).wait()
                pltpu.sync_copy(bufs[b], s_sp.at[dst_v.at[j]], add=True)
                pltpu.async_copy(u_hbm.at[src_v.at[j + NBUF]], bufs[b],
                                 sems[b])
            return cr

        lax.fori_loop(0, CBT - 1, blk, 0, unroll=False)
        base = (CBT - 1) * NBUF
        for b in range(NBUF):
            j = base + b
            pltpu.make_async_copy(u_hbm.at[src_v.at[j]], bufs[b],
                                  sems[b]).wait()
            pltpu.sync_copy(bufs[b], s_sp.at[dst_v.at[j]], add=True)

    def _idx_start(q, sv, dv, sa, sb):
        pltpu.async_copy(src_hbm.at[c, s, pl.ds(q * CB, CB)], sv, sa)
        pltpu.async_copy(dst_hbm.at[s, pl.ds(q * CB, CB)], dv, sb)

    def _idx_wait(q, sv, dv, sa, sb):
        pltpu.make_async_copy(src_hbm.at[c, s, pl.ds(q * CB, CB)], sv,
                              sa).wait()
        pltpu.make_async_copy(dst_hbm.at[s, pl.ds(q * CB, CB)], dv,
                              sb).wait()

    def hop(k, carry):
        pltpu.sync_copy(src_hbm.at[c, s, pl.ds(0, CB)], src_va)
        pltpu.sync_copy(dst_hbm.at[s, pl.ds(0, CB)], dst_va)

        def chunk2(t, cq):
            q0 = 2 * t
            _idx_start(q0 + 1, src_vb, dst_vb, mia, mib)
            _proc(src_va, dst_va)
            _idx_wait(q0 + 1, src_vb, dst_vb, mia, mib)

            @pl.when(t < NCB2 - 1)
            def _():
                _idx_start(q0 + 2, src_va, dst_va, mia, mib)

            _proc(src_vb, dst_vb)

            @pl.when(t < NCB2 - 1)
            def _():
                _idx_wait(q0 + 2, src_va, dst_va, mia, mib)

            return cq

        lax.fori_loop(0, NCB2, chunk2, 0, unroll=False)
        plsc.subcore_barrier()

        pltpu.sync_copy(s_sp.at[pl.ds(r0, RPT)], strip_v)
        last = k == K - 1

        def comb(r, cr):
            d = dinv_v[r]
            s0 = strip_v[r, pl.ds(0, 16)]
            s1 = strip_v[r, pl.ds(16, 16)]
            e0 = c2_v[r, pl.ds(0, 16)]
            e1 = c2_v[r, pl.ds(16, 16)]
            ds0 = (1.0 - ALPHA) * d * s0
            ds1 = (1.0 - ALPHA) * d * s1
            strip_v[r, pl.ds(0, 16)] = jnp.where(last, ds0 + e0,
                                                 d * (ds0 + e0))
            strip_v[r, pl.ds(16, 16)] = jnp.where(last, ds1 + e1,
                                                  d * (ds1 + e1))
            return cr

        lax.fori_loop(0, RPT, comb, 0, unroll=False)

        pltpu.sync_copy(strip_v, u_hbm.at[pl.ds(ubase, RPT)])

        @pl.when(last)
        def _():
            pltpu.sync_copy(strip_v,
                            out_hbm.at[pl.ds(r0, RPT), pl.ds(c * HALF, HALF)])

        pltpu.sync_copy(strip_v, s_sp.at[pl.ds(r0, RPT)])
        plsc.subcore_barrier()
        return carry

    lax.fori_loop(0, K, hop, 0, unroll=False)


@functools.partial(
    pl.kernel,
    out_type=[
        jax.ShapeDtypeStruct((2 * N_PAD, HALF), jnp.float32),
        jax.ShapeDtypeStruct((N_PAD, 2 * HALF), jnp.float32),
    ],
    mesh=_mesh,
    compiler_params=pltpu.CompilerParams(use_tc_tiling_on_sc=False),
    scratch_types=[
        pltpu.VMEM((CB, B), jnp.int32),
        pltpu.VMEM((CB, B), jnp.int32),
        pltpu.VMEM((CB, B), jnp.int32),
        pltpu.VMEM((CB, B), jnp.int32),
        pltpu.VMEM((B, HALF), jnp.float32),
        pltpu.VMEM((B, HALF), jnp.float32),
        pltpu.VMEM((B, HALF), jnp.float32),
        pltpu.VMEM((B, HALF), jnp.float32),
        pltpu.VMEM((B, HALF), jnp.float32),
        pltpu.VMEM((B, HALF), jnp.float32),
        pltpu.VMEM((B, HALF), jnp.float32),
        pltpu.VMEM((B, HALF), jnp.float32),
        pltpu.VMEM((B, HALF), jnp.float32),
        pltpu.VMEM((RPT, 16), jnp.float32),
        pltpu.VMEM((RPT, HALF), jnp.float32),
        pltpu.VMEM((RPT, HALF), jnp.float32),
        pltpu.VMEM_SHARED((N_PAD, HALF), jnp.float32),
        pltpu.SemaphoreType.DMA,
        pltpu.SemaphoreType.DMA,
        pltpu.SemaphoreType.DMA,
        pltpu.SemaphoreType.DMA,
        pltpu.SemaphoreType.DMA,
        pltpu.SemaphoreType.DMA,
        pltpu.SemaphoreType.DMA,
        pltpu.SemaphoreType.DMA,
        pltpu.SemaphoreType.DMA,
        pltpu.SemaphoreType.DMA,
    ],
)
def _appnp_sc(h_hbm, src_hbm, dst_hbm, u_hbm, out_hbm,
              src_va, dst_va, src_vb, dst_vb,
              b0, b1, b2, b3, b4, b5, b6, b7,
              ones_v, dinv_v, c2_v, strip_v, s_sp,
              m0, m1, m2, m3, m4, m5, m6, m7, mia, mib):
    _appnp_body(h_hbm, src_hbm, dst_hbm, u_hbm, out_hbm,
                src_va, dst_va, src_vb, dst_vb,
                b0, b1, b2, b3, b4, b5, b6, b7,
                ones_v, dinv_v, c2_v, strip_v, s_sp,
                m0, m1, m2, m3, m4, m5, m6, m7, mia, mib)


def kernel(x, edge_index, W1, b1, W2, b2):
    src_p = edge_index[0].reshape(16, NB, B)
    dst_p = edge_index[1].reshape(16, NB, B)
    src_b = jnp.stack([src_p, src_p + N_PAD])

    x_pad = jnp.concatenate([x, jnp.zeros((N_PAD - N, 128), jnp.float32)])

    w2s = W2.reshape(2, HALF, 128)
    b2s = b2.reshape(2, 1, HALF)
    h2 = _mlp(x_pad, W1, b1.reshape(1, -1), w2s, b2s)

    _, outp = _appnp_sc(h2, src_b, dst_p)
    return outp[:N]


# speedup vs baseline: 1.4644x; 1.0152x over previous
import functools

import jax
import jax.numpy as jnp
from jax import lax
from jax.experimental import pallas as pl
from jax.experimental.pallas import tpu as pltpu
from jax.experimental.pallas import tpu_sc as plsc

N = 10000
N_PAD = 10240
E = 320000
HALF = 32
K = 5
ALPHA = 0.1
B = 125
NBUF = 8
NB = 160
CB = 40
NCB = NB // CB
CBT = CB // NBUF
RPT = N_PAD // 16

_mesh = plsc.VectorSubcoreMesh(core_axis_name="c", subcore_axis_name="s")


def _mlp_body(x_ref, w1_ref, b1_ref, w2_ref, b2_ref, o_ref):
    dn = (((1,), (1,)), ((), ()))
    h = jnp.maximum(
        lax.dot_general(x_ref[...], w1_ref[...], dn,
                        preferred_element_type=jnp.float32) + b1_ref[...],
        0.0,
    )
    o_ref[0] = (
        lax.dot_general(h, w2_ref[0], dn,
                        preferred_element_type=jnp.float32) + b2_ref[0]
    )


def _mlp(x_pad, w1t, b1, w2t, b2):
    blk = 1024
    return pl.pallas_call(
        _mlp_body,
        grid=(2, N_PAD // blk),
        in_specs=[
            pl.BlockSpec((blk, 128), lambda c, i: (i, 0)),
            pl.BlockSpec((128, 128), lambda c, i: (0, 0)),
            pl.BlockSpec((1, 128), lambda c, i: (0, 0)),
            pl.BlockSpec((1, HALF, 128), lambda c, i: (c, 0, 0)),
            pl.BlockSpec((1, 1, HALF), lambda c, i: (c, 0, 0)),
        ],
        out_specs=pl.BlockSpec((1, blk, HALF), lambda c, i: (c, i, 0)),
        out_shape=jax.ShapeDtypeStruct((2, N_PAD, HALF), jnp.float32),
    )(x_pad, w1t, b1, w2t, b2)


def _rsqrt16(x):
    i = lax.bitcast_convert_type(x, jnp.int32)
    i = jnp.int32(0x5F3759DF) - (i >> 1)
    y = lax.bitcast_convert_type(i, jnp.float32)
    for _ in range(4):
        y = y * (1.5 - 0.5 * x * y * y)
    return y


def _appnp_body(h_hbm, src_hbm, dst_hbm, u_hbm, out_hbm,
                src_va, dst_va, src_vb, dst_vb,
                b0, b1, b2, b3, b4, b5, b6, b7,
                ones_v, dinv_v, c2_v, strip_v, s_sp,
                m0, m1, m2, m3, m4, m5, m6, m7, mia, mib):
    bufs = (b0, b1, b2, b3, b4, b5, b6, b7)
    sems = (m0, m1, m2, m3, m4, m5, m6, m7)
    c = lax.axis_index("c")
    s = lax.axis_index("s")
    r0 = s * RPT
    ubase = c * N_PAD + r0

    one16 = jnp.full((16,), 1.0, jnp.float32)
    zero16 = jnp.zeros((16,), jnp.float32)

    def fill_ones(j, carry):
        ones_v[j, pl.ds(0, 16)] = one16
        ones_v[j, pl.ds(16, 16)] = one16
        return carry

    lax.fori_loop(0, B, fill_ones, 0, unroll=False)

    def zero_strip(r, carry):
        strip_v[r, pl.ds(0, 16)] = zero16
        strip_v[r, pl.ds(16, 16)] = zero16
        return carry

    lax.fori_loop(0, RPT, zero_strip, 0, unroll=False)

    pltpu.sync_copy(strip_v, s_sp.at[pl.ds(r0, RPT)])
    plsc.subcore_barrier()

    def degchunk(q, carry):
        pltpu.sync_copy(dst_hbm.at[s, pl.ds(q * CB, CB)], dst_va)

        def degblk(j, cr):
            pltpu.sync_copy(ones_v, s_sp.at[dst_va.at[j]], add=True)
            return cr

        lax.fori_loop(0, CB, degblk, 0, unroll=False)
        return carry

    lax.fori_loop(0, NCB, degchunk, 0, unroll=False)
    plsc.subcore_barrier()

    pltpu.sync_copy(s_sp.at[pl.ds(r0, RPT)], strip_v)

    pltpu.sync_copy(h_hbm.at[c, pl.ds(r0, RPT)], c2_v)

    def setup_row(r, carry):
        d = _rsqrt16(strip_v[r, pl.ds(0, 16)] + 1.0)
        dinv_v[r] = d
        h0 = c2_v[r, pl.ds(0, 16)]
        h1 = c2_v[r, pl.ds(16, 16)]
        c2_v[r, pl.ds(0, 16)] = ALPHA * h0
        c2_v[r, pl.ds(16, 16)] = ALPHA * h1
        strip_v[r, pl.ds(0, 16)] = d * h0
        strip_v[r, pl.ds(16, 16)] = d * h1
        return carry

    lax.fori_loop(0, RPT, setup_row, 0, unroll=False)
    pltpu.sync_copy(strip_v, u_hbm.at[pl.ds(ubase, RPT)])
    pltpu.sync_copy(strip_v, s_sp.at[pl.ds(r0, RPT)])
    plsc.subcore_barrier()

    NCB2 = NCB // 2

    def _proc(src_v, dst_v):
        for b in range(NBUF):
            pltpu.async_copy(u_hbm.at[src_v.at[b]], bufs[b], sems[b])

        def blk(t, cr):
            base = t * NBUF
            for b in range(NBUF):
                j = base + b
                pltpu.make_async_copy(u_hbm.at[src_v.at[j]], bufs[b],
                                      sems[b]).wait()
                pltpu.sync_copy(bufs[b], s_sp.at[dst_v.at[j]], add=True)
                pltpu.async_copy(u_hbm.at[src_v.at[j + NBUF]], bufs[b],
                                 sems[b])
            return cr

        lax.fori_loop(0, CBT - 1, blk, 0, unroll=False)
        base = (CBT - 1) * NBUF
        for b in range(NBUF):
            j = base + b
            pltpu.make_async_copy(u_hbm.at[src_v.at[j]], bufs[b],
                                  sems[b]).wait()
            pltpu.sync_copy(bufs[b], s_sp.at[dst_v.at[j]], add=True)

    def _idx_start(q, sv, dv, sa, sb):
        pltpu.async_copy(src_hbm.at[c, s, pl.ds(q * CB, CB)], sv, sa)
        pltpu.async_copy(dst_hbm.at[s, pl.ds(q * CB, CB)], dv, sb)

    def _idx_wait(q, sv, dv, sa, sb):
        pltpu.make_async_copy(src_hbm.at[c, s, pl.ds(q * CB, CB)], sv,
                              sa).wait()
        pltpu.make_async_copy(dst_hbm.at[s, pl.ds(q * CB, CB)], dv,
                              sb).wait()

    pltpu.sync_copy(src_hbm.at[c, s, pl.ds(0, CB)], src_va)
    pltpu.sync_copy(dst_hbm.at[s, pl.ds(0, CB)], dst_va)

    def hop(k, carry):
        def chunk2(t, cq):
            q0 = 2 * t
            _idx_start(q0 + 1, src_vb, dst_vb, mia, mib)
            _proc(src_va, dst_va)
            _idx_wait(q0 + 1, src_vb, dst_vb, mia, mib)
            qn = jnp.where(t < NCB2 - 1, q0 + 2, 0)
            _idx_start(qn, src_va, dst_va, mia, mib)
            _proc(src_vb, dst_vb)
            _idx_wait(qn, src_va, dst_va, mia, mib)
            return cq

        lax.fori_loop(0, NCB2, chunk2, 0, unroll=False)
        plsc.subcore_barrier()

        pltpu.sync_copy(s_sp.at[pl.ds(r0, RPT)], strip_v)
        last = k == K - 1

        def comb(r, cr):
            d = dinv_v[r]
            s0 = strip_v[r, pl.ds(0, 16)]
            s1 = strip_v[r, pl.ds(16, 16)]
            e0 = c2_v[r, pl.ds(0, 16)]
            e1 = c2_v[r, pl.ds(16, 16)]
            ds0 = (1.0 - ALPHA) * d * s0
            ds1 = (1.0 - ALPHA) * d * s1
            strip_v[r, pl.ds(0, 16)] = jnp.where(last, ds0 + e0,
                                                 d * (ds0 + e0))
            strip_v[r, pl.ds(16, 16)] = jnp.where(last, ds1 + e1,
                                                  d * (ds1 + e1))
            return cr

        lax.fori_loop(0, RPT, comb, 0, unroll=False)

        pltpu.sync_copy(strip_v, u_hbm.at[pl.ds(ubase, RPT)])

        @pl.when(last)
        def _():
            pltpu.sync_copy(strip_v,
                            out_hbm.at[pl.ds(r0, RPT), pl.ds(c * HALF, HALF)])

        pltpu.sync_copy(strip_v, s_sp.at[pl.ds(r0, RPT)])
        plsc.subcore_barrier()
        return carry

    lax.fori_loop(0, K, hop, 0, unroll=False)


@functools.partial(
    pl.kernel,
    out_type=[
        jax.ShapeDtypeStruct((2 * N_PAD, HALF), jnp.float32),
        jax.ShapeDtypeStruct((N_PAD, 2 * HALF), jnp.float32),
    ],
    mesh=_mesh,
    compiler_params=pltpu.CompilerParams(use_tc_tiling_on_sc=False),
    scratch_types=[
        pltpu.VMEM((CB, B), jnp.int32),
        pltpu.VMEM((CB, B), jnp.int32),
        pltpu.VMEM((CB, B), jnp.int32),
        pltpu.VMEM((CB, B), jnp.int32),
        pltpu.VMEM((B, HALF), jnp.float32),
        pltpu.VMEM((B, HALF), jnp.float32),
        pltpu.VMEM((B, HALF), jnp.float32),
        pltpu.VMEM((B, HALF), jnp.float32),
        pltpu.VMEM((B, HALF), jnp.float32),
        pltpu.VMEM((B, HALF), jnp.float32),
        pltpu.VMEM((B, HALF), jnp.float32),
        pltpu.VMEM((B, HALF), jnp.float32),
        pltpu.VMEM((B, HALF), jnp.float32),
        pltpu.VMEM((RPT, 16), jnp.float32),
        pltpu.VMEM((RPT, HALF), jnp.float32),
        pltpu.VMEM((RPT, HALF), jnp.float32),
        pltpu.VMEM_SHARED((N_PAD, HALF), jnp.float32),
        pltpu.SemaphoreType.DMA,
        pltpu.SemaphoreType.DMA,
        pltpu.SemaphoreType.DMA,
        pltpu.SemaphoreType.DMA,
        pltpu.SemaphoreType.DMA,
        pltpu.SemaphoreType.DMA,
        pltpu.SemaphoreType.DMA,
        pltpu.SemaphoreType.DMA,
        pltpu.SemaphoreType.DMA,
        pltpu.SemaphoreType.DMA,
    ],
)
def _appnp_sc(h_hbm, src_hbm, dst_hbm, u_hbm, out_hbm,
              src_va, dst_va, src_vb, dst_vb,
              b0, b1, b2, b3, b4, b5, b6, b7,
              ones_v, dinv_v, c2_v, strip_v, s_sp,
              m0, m1, m2, m3, m4, m5, m6, m7, mia, mib):
    _appnp_body(h_hbm, src_hbm, dst_hbm, u_hbm, out_hbm,
                src_va, dst_va, src_vb, dst_vb,
                b0, b1, b2, b3, b4, b5, b6, b7,
                ones_v, dinv_v, c2_v, strip_v, s_sp,
                m0, m1, m2, m3, m4, m5, m6, m7, mia, mib)


def kernel(x, edge_index, W1, b1, W2, b2):
    src_p = edge_index[0].reshape(16, NB, B)
    dst_p = edge_index[1].reshape(16, NB, B)
    src_b = jnp.stack([src_p, src_p + N_PAD])

    x_pad = jnp.concatenate([x, jnp.zeros((N_PAD - N, 128), jnp.float32)])

    w2s = W2.reshape(2, HALF, 128)
    b2s = b2.reshape(2, 1, HALF)
    h2 = _mlp(x_pad, W1, b1.reshape(1, -1), w2s, b2s)

    _, outp = _appnp_sc(h2, src_b, dst_p)
    return outp[:N]
